# Initial kernel scaffold; baseline (speedup 1.0000x reference)
import jax
import jax.numpy as jnp
from jax.experimental import pallas as pl


def kernel(input, vectors, bias):
    raise NotImplementedError("write your pallas kernel here")


# speedup vs baseline: 7.9071x; 7.9071x over previous
import functools

import jax
import jax.numpy as jnp
import numpy as np
from jax import lax
from jax.experimental import pallas as pl
from jax.experimental.pallas import tpu as pltpu
from jax.experimental.pallas import tpu_sc as plsc

_MAPPING = np.array([0, 1, 1, 1, 1, 2, 2, 2, 2, 3, 3, 3, 3, 3, 4, 4, 4, 4, 5, 6, 7, 7, 7],
                    dtype=np.int32)
_DISTANCES = np.array([0, 0, 1, 2, 3, 0, 1, 2, 3, 0, 1, 2, 3, 4, 0, 1, 2, 3, 0, 0, 0, 1, 2],
                      dtype=np.float32)
_NCODE = 23
_NMAJOR = 8
_EMBED = 32
_TROWS = 32

_OH = np.zeros((_TROWS, _NMAJOR), np.float32)
_OH[np.arange(_NCODE), _MAPPING] = 1.0
_SOH = _OH * np.pad(_DISTANCES, (0, _TROWS - _NCODE))[:, None]

_B = 16 * 1 * 224 * 224
_NW = 32
_BPW = _B // _NW
_CH = 1792
_NCHUNK = _BPW // _CH


def _table_body(vec_ref, bias_ref, oh_ref, soh_ref, tab_ref):
    tab_ref[:, :] = (
        jnp.dot(oh_ref[:, :], bias_ref[:, :], preferred_element_type=jnp.float32,
                precision=jax.lax.Precision.HIGHEST)
        + jnp.dot(soh_ref[:, :], vec_ref[:, :], preferred_element_type=jnp.float32,
                  precision=jax.lax.Precision.HIGHEST)
    )


def _build_table(vectors, bias):
    return pl.pallas_call(
        _table_body,
        out_shape=jax.ShapeDtypeStruct((_TROWS, _EMBED), jnp.float32),
    )(vectors, bias, jnp.asarray(_OH), jnp.asarray(_SOH))


_mesh = plsc.VectorSubcoreMesh(core_axis_name="c", subcore_axis_name="s")


@functools.partial(
    pl.kernel,
    mesh=_mesh,
    out_type=jax.ShapeDtypeStruct((_B, _EMBED), jnp.float32),
    scratch_types=[
        pltpu.VMEM((_CH,), jnp.int32),
        pltpu.VMEM((_CH, _EMBED), jnp.float32),
        pltpu.SemaphoreType.DMA,
    ],
    compiler_params=pltpu.CompilerParams(use_tc_tiling_on_sc=False),
)
def _gather_kernel(idx_hbm, tab_hbm, out_hbm, idx_v, rows_v, sem):
    wid = lax.axis_index("s") * 2 + lax.axis_index("c")
    base = wid * _BPW

    def body(j, carry):
        off = base + j * _CH
        pltpu.sync_copy(idx_hbm.at[pl.ds(off, _CH)], idx_v)
        pltpu.async_copy(tab_hbm.at[idx_v], rows_v, sem).wait()
        pltpu.sync_copy(rows_v, out_hbm.at[pl.ds(off, _CH), :])
        return carry

    lax.fori_loop(0, _NCHUNK, body, 0)


def kernel(input, vectors, bias):
    table = _build_table(vectors, bias)
    idx = input.reshape(_B)
    out = _gather_kernel(idx, table)
    return out.reshape(input.shape + (_EMBED,))


# speedup vs baseline: 31.2987x; 3.9583x over previous
import functools

import jax
import jax.numpy as jnp
import numpy as np
from jax import lax
from jax.experimental import pallas as pl
from jax.experimental.pallas import tpu as pltpu
from jax.experimental.pallas import tpu_sc as plsc

_MAPPING = np.array([0, 1, 1, 1, 1, 2, 2, 2, 2, 3, 3, 3, 3, 3, 4, 4, 4, 4, 5, 6, 7, 7, 7],
                    dtype=np.int32)
_DISTANCES = np.array([0, 0, 1, 2, 3, 0, 1, 2, 3, 0, 1, 2, 3, 4, 0, 1, 2, 3, 0, 0, 0, 1, 2],
                      dtype=np.float32)
_NCODE = 23
_NMAJOR = 8
_EMBED = 32
_TROWS = 32

_OH = np.zeros((_TROWS, _NMAJOR), np.float32)
_OH[np.arange(_NCODE), _MAPPING] = 1.0
_SOH = _OH * np.pad(_DISTANCES, (0, _TROWS - _NCODE))[:, None]

_B = 16 * 1 * 224 * 224
_NW = 32
_BPW = _B // _NW
_CH = 1568
_NCHUNK = _BPW // _CH


def _table_body(vec_ref, bias_ref, oh_ref, soh_ref, tab_ref):
    tab_ref[:, :] = (
        jnp.dot(oh_ref[:, :], bias_ref[:, :], preferred_element_type=jnp.float32,
                precision=jax.lax.Precision.HIGHEST)
        + jnp.dot(soh_ref[:, :], vec_ref[:, :], preferred_element_type=jnp.float32,
                  precision=jax.lax.Precision.HIGHEST)
    )


def _build_table(vectors, bias):
    return pl.pallas_call(
        _table_body,
        out_shape=jax.ShapeDtypeStruct((_TROWS, _EMBED), jnp.float32),
    )(vectors, bias, jnp.asarray(_OH), jnp.asarray(_SOH))


_mesh = plsc.VectorSubcoreMesh(core_axis_name="c", subcore_axis_name="s")


@functools.partial(
    pl.kernel,
    mesh=_mesh,
    out_type=jax.ShapeDtypeStruct((_B, _EMBED), jnp.float32),
    scratch_types=[
        pltpu.VMEM((_BPW,), jnp.int32),
        pltpu.VMEM((2, _CH, _EMBED), jnp.float32),
        pltpu.VMEM_SHARED((_TROWS, _EMBED), jnp.float32),
        pltpu.SemaphoreType.DMA,
        pltpu.SemaphoreType.DMA,
        pltpu.SemaphoreType.DMA,
    ],
    compiler_params=pltpu.CompilerParams(use_tc_tiling_on_sc=False),
)
def _gather_kernel(idx_hbm, tab_hbm, out_hbm, idx_v, rows_v, tab_sh, gsem, ssem0, ssem1):
    cid = lax.axis_index("c")
    sid = lax.axis_index("s")
    wid = sid * 2 + cid
    base = wid * _BPW

    @pl.when(sid == 0)
    def _():
        pltpu.sync_copy(tab_hbm, tab_sh)

    plsc.subcore_barrier()

    pltpu.sync_copy(idx_hbm.at[pl.ds(base, _BPW)], idx_v)

    ssems = (ssem0, ssem1)
    handles = [None, None]
    for j in range(_NCHUNK):
        b = j % 2
        if handles[b] is not None:
            handles[b].wait()
        pltpu.async_copy(
            tab_sh.at[idx_v.at[pl.ds(j * _CH, _CH)]], rows_v.at[b], gsem
        ).wait()
        handles[b] = pltpu.async_copy(
            rows_v.at[b], out_hbm.at[pl.ds(base + j * _CH, _CH)], ssems[b]
        )
    handles[0].wait()
    handles[1].wait()


def kernel(input, vectors, bias):
    table = _build_table(vectors, bias)
    idx = input.reshape(_B)
    out = _gather_kernel(idx, table)
    return out.reshape(input.shape + (_EMBED,))
